# baseline (speedup 1.0000x reference)
import functools

import jax
import jax.numpy as jnp
from jax import lax
from jax.experimental import pallas as pl
from jax.experimental.pallas import tpu as pltpu
from jax.experimental.pallas import tpu_sc as plsc

HIDDEN = 4096
NUM_EXPERTS = 64
TOP_K = 8
BLOCK_T = 512

_NC = 2
_NS = 16
_NW = _NC * _NS
_LANES = 16

_SC_CHUNK = 256


def _matmul_body(h_ref, wt_ref, logits_ref):
    logits_ref[...] = jnp.dot(h_ref[...], wt_ref[...],
                              preferred_element_type=jnp.float32)


def _tc_logits(h_flat, wt):
    n_tokens = h_flat.shape[0]
    return pl.pallas_call(
        _matmul_body,
        grid=(n_tokens // BLOCK_T,),
        in_specs=[
            pl.BlockSpec((BLOCK_T, HIDDEN), lambda i: (i, 0)),
            pl.BlockSpec((HIDDEN, NUM_EXPERTS), lambda i: (0, 0)),
        ],
        out_specs=pl.BlockSpec((BLOCK_T, NUM_EXPERTS), lambda i: (i, 0)),
        out_shape=jax.ShapeDtypeStruct((n_tokens, NUM_EXPERTS), jnp.float32),
        compiler_params=pltpu.CompilerParams(
            dimension_semantics=("arbitrary",),
        ),
    )(h_flat, wt)


def _sc_topk_body(logits_hbm, vals_hbm, idx_hbm, chunk, outv, outi):
    n_tokens = logits_hbm.shape[0] // NUM_EXPERTS
    per_worker = n_tokens // _NW
    n_chunks = per_worker // _SC_CHUNK
    wid = lax.axis_index("c") * _NS + lax.axis_index("s")
    lane = lax.iota(jnp.int32, _LANES)
    neg_inf = jnp.full((_LANES,), -jnp.inf, jnp.float32)
    ones = jnp.ones((_LANES,), jnp.int32)

    def process_group(g, _):
        rows = g * _LANES + lane
        rows_e = rows * NUM_EXPERTS
        rows_k = rows * TOP_K
        best_vals = []
        best_idxs = []
        for _k in range(TOP_K):
            fi = rows_e
            bv = plsc.load_gather(chunk, [fi])
            bi = jnp.zeros((_LANES,), jnp.int32)
            for e in range(1, NUM_EXPERTS):
                fi = fi + ones
                v = plsc.load_gather(chunk, [fi])
                gt = v > bv
                bv = jnp.where(gt, v, bv)
                bi = jnp.where(gt, jnp.full((_LANES,), e, jnp.int32), bi)
            plsc.store_scatter(chunk, [rows_e + bi], neg_inf)
            best_vals.append(bv)
            best_idxs.append(bi)
        exps = [jnp.exp(v - best_vals[0]) for v in best_vals]
        denom = exps[0]
        for e_ in exps[1:]:
            denom = denom + e_
        for k in range(TOP_K):
            fo = rows_k + k
            plsc.store_scatter(outv, [fo], exps[k] / denom)
            plsc.store_scatter(outi, [fo], best_idxs[k])
        return _

    def process_chunk(c, _):
        base = wid * per_worker + c * _SC_CHUNK
        pltpu.sync_copy(logits_hbm.at[pl.ds(base * NUM_EXPERTS,
                                            _SC_CHUNK * NUM_EXPERTS)], chunk)
        lax.fori_loop(0, _SC_CHUNK // _LANES, process_group, None)
        pltpu.sync_copy(outv, vals_hbm.at[pl.ds(base * TOP_K,
                                                _SC_CHUNK * TOP_K)])
        pltpu.sync_copy(outi, idx_hbm.at[pl.ds(base * TOP_K,
                                               _SC_CHUNK * TOP_K)])
        return _

    lax.fori_loop(0, n_chunks, process_chunk, None)


def _sc_topk(logits):
    n_tokens = logits.shape[0]
    mesh = plsc.VectorSubcoreMesh(core_axis_name="c", subcore_axis_name="s")
    vals, idx = pl.kernel(
        _sc_topk_body,
        out_type=[
            jax.ShapeDtypeStruct((n_tokens * TOP_K,), jnp.float32),
            jax.ShapeDtypeStruct((n_tokens * TOP_K,), jnp.int32),
        ],
        mesh=mesh,
        compiler_params=pltpu.CompilerParams(needs_layout_passes=False),
        scratch_types=[
            pltpu.VMEM((_SC_CHUNK * NUM_EXPERTS,), jnp.float32),
            pltpu.VMEM((_SC_CHUNK * TOP_K,), jnp.float32),
            pltpu.VMEM((_SC_CHUNK * TOP_K,), jnp.int32),
        ],
    )(logits.reshape(-1))
    return vals.reshape(n_tokens, TOP_K), idx.reshape(n_tokens, TOP_K)


@jax.jit
def kernel(hidden_states, weight):
    h_flat = hidden_states.reshape(-1, hidden_states.shape[-1])
    wt = weight.T
    logits = _tc_logits(h_flat, wt)
    vals, idx = _sc_topk(logits)
    return (logits, vals.astype(hidden_states.dtype), idx)


# speedup vs baseline: 1.0207x; 1.0207x over previous
import functools

import jax
import jax.numpy as jnp
from jax import lax
from jax.experimental import pallas as pl
from jax.experimental.pallas import tpu as pltpu
from jax.experimental.pallas import tpu_sc as plsc

HIDDEN = 4096
NUM_EXPERTS = 64
TOP_K = 8
BLOCK_T = 512

_NC = 2
_NS = 16
_NW = _NC * _NS
_LANES = 16

_SC_CHUNK = 256


def _matmul_body(h_ref, wt_ref, logits_ref):
    logits_ref[...] = jnp.dot(h_ref[...], wt_ref[...],
                              preferred_element_type=jnp.float32)


def _tc_logits(h_flat, wt):
    n_tokens = h_flat.shape[0]
    return pl.pallas_call(
        _matmul_body,
        grid=(n_tokens // BLOCK_T,),
        in_specs=[
            pl.BlockSpec((BLOCK_T, HIDDEN), lambda i: (i, 0)),
            pl.BlockSpec((HIDDEN, NUM_EXPERTS), lambda i: (0, 0)),
        ],
        out_specs=pl.BlockSpec((BLOCK_T, NUM_EXPERTS), lambda i: (i, 0)),
        out_shape=jax.ShapeDtypeStruct((n_tokens, NUM_EXPERTS), jnp.float32),
        compiler_params=pltpu.CompilerParams(
            dimension_semantics=("arbitrary",),
        ),
    )(h_flat, wt)


def _sc_topk_body(logits_hbm, vals_hbm, idx_hbm, chunk, outv, outi):
    n_tokens = logits_hbm.shape[0] // NUM_EXPERTS
    per_worker = n_tokens // _NW
    n_chunks = per_worker // _SC_CHUNK
    wid = lax.axis_index("c") * _NS + lax.axis_index("s")
    lane = lax.iota(jnp.int32, _LANES)
    neg_inf = jnp.full((_LANES,), -jnp.inf, jnp.float32)
    ones = jnp.ones((_LANES,), jnp.int32)

    def argmax_tree(pairs):
        while len(pairs) > 1:
            nxt = []
            for i in range(0, len(pairs) - 1, 2):
                (va, ia), (vb, ib) = pairs[i], pairs[i + 1]
                gt = vb > va
                nxt.append((jnp.where(gt, vb, va), jnp.where(gt, ib, ia)))
            if len(pairs) % 2:
                nxt.append(pairs[-1])
            pairs = nxt
        return pairs[0]

    def process_group(g, _):
        rows = g * _LANES + lane
        rows_e = rows * NUM_EXPERTS
        rows_k = rows * TOP_K
        best_vals = []
        best_idxs = []
        for _k in range(TOP_K):
            groups = []
            for c in range(0, NUM_EXPERTS, 8):
                leaves = []
                for e in range(c, c + 8):
                    v = plsc.load_gather(chunk, [rows_e + e])
                    leaves.append((v, jnp.full((_LANES,), e, jnp.int32)))
                groups.append(argmax_tree(leaves))
            bv, bi = argmax_tree(groups)
            plsc.store_scatter(chunk, [rows_e + bi], neg_inf)
            best_vals.append(bv)
            best_idxs.append(bi)
        exps = [jnp.exp(v - best_vals[0]) for v in best_vals]
        denom = exps[0]
        for e_ in exps[1:]:
            denom = denom + e_
        for k in range(TOP_K):
            fo = rows_k + k
            plsc.store_scatter(outv, [fo], exps[k] / denom)
            plsc.store_scatter(outi, [fo], best_idxs[k])
        return _

    def process_chunk(c, _):
        base = wid * per_worker + c * _SC_CHUNK
        pltpu.sync_copy(logits_hbm.at[pl.ds(base * NUM_EXPERTS,
                                            _SC_CHUNK * NUM_EXPERTS)], chunk)
        lax.fori_loop(0, _SC_CHUNK // _LANES, process_group, None)
        pltpu.sync_copy(outv, vals_hbm.at[pl.ds(base * TOP_K,
                                                _SC_CHUNK * TOP_K)])
        pltpu.sync_copy(outi, idx_hbm.at[pl.ds(base * TOP_K,
                                               _SC_CHUNK * TOP_K)])
        return _

    lax.fori_loop(0, n_chunks, process_chunk, None)


def _sc_topk(logits):
    n_tokens = logits.shape[0]
    mesh = plsc.VectorSubcoreMesh(core_axis_name="c", subcore_axis_name="s")
    vals, idx = pl.kernel(
        _sc_topk_body,
        out_type=[
            jax.ShapeDtypeStruct((n_tokens * TOP_K,), jnp.float32),
            jax.ShapeDtypeStruct((n_tokens * TOP_K,), jnp.int32),
        ],
        mesh=mesh,
        compiler_params=pltpu.CompilerParams(needs_layout_passes=False),
        scratch_types=[
            pltpu.VMEM((_SC_CHUNK * NUM_EXPERTS,), jnp.float32),
            pltpu.VMEM((_SC_CHUNK * TOP_K,), jnp.float32),
            pltpu.VMEM((_SC_CHUNK * TOP_K,), jnp.int32),
        ],
    )(logits.reshape(-1))
    return vals.reshape(n_tokens, TOP_K), idx.reshape(n_tokens, TOP_K)


@jax.jit
def kernel(hidden_states, weight):
    h_flat = hidden_states.reshape(-1, hidden_states.shape[-1])
    wt = weight.T
    logits = _tc_logits(h_flat, wt)
    vals, idx = _sc_topk(logits)
    return (logits, vals.astype(hidden_states.dtype), idx)


# speedup vs baseline: 1.8379x; 1.8007x over previous
import functools

import jax
import jax.numpy as jnp
from jax import lax
from jax.experimental import pallas as pl
from jax.experimental.pallas import tpu as pltpu
from jax.experimental.pallas import tpu_sc as plsc

HIDDEN = 4096
NUM_EXPERTS = 64
TOP_K = 8
BLOCK_T = 512

_NC = 2
_NS = 16
_NW = _NC * _NS
_LANES = 16

_SC_CHUNK = 512


def _matmul_body(h_ref, wt_ref, logits_ref, logits_t_ref):
    logits = jnp.dot(h_ref[...], wt_ref[...],
                     preferred_element_type=jnp.float32)
    logits_ref[...] = logits
    logits_t_ref[...] = jnp.swapaxes(logits, 0, 1)


def _tc_logits(h_flat, wt):
    n_tokens = h_flat.shape[0]
    return pl.pallas_call(
        _matmul_body,
        grid=(n_tokens // BLOCK_T,),
        in_specs=[
            pl.BlockSpec((BLOCK_T, HIDDEN), lambda i: (i, 0)),
            pl.BlockSpec((HIDDEN, NUM_EXPERTS), lambda i: (0, 0)),
        ],
        out_specs=[
            pl.BlockSpec((BLOCK_T, NUM_EXPERTS), lambda i: (i, 0)),
            pl.BlockSpec((NUM_EXPERTS, BLOCK_T), lambda i: (0, i)),
        ],
        out_shape=[
            jax.ShapeDtypeStruct((n_tokens, NUM_EXPERTS), jnp.float32),
            jax.ShapeDtypeStruct((NUM_EXPERTS, n_tokens), jnp.float32),
        ],
        compiler_params=pltpu.CompilerParams(
            dimension_semantics=("arbitrary",),
        ),
    )(h_flat, wt)


def _sc_topk_body(logits_t_hbm, vals_hbm, idx_hbm, tr, outv, outi):
    n_tokens = logits_t_hbm.shape[1]
    per_worker = n_tokens // _NW
    n_chunks = per_worker // _SC_CHUNK
    wid = lax.axis_index("c") * _NS + lax.axis_index("s")
    lane = lax.iota(jnp.int32, _LANES)
    neg_inf = jnp.full((_LANES,), -jnp.inf, jnp.float32)

    def argmax_tree(pairs):
        while len(pairs) > 1:
            nxt = []
            for i in range(0, len(pairs) - 1, 2):
                (va, ia), (vb, ib) = pairs[i], pairs[i + 1]
                gt = vb > va
                nxt.append((jnp.where(gt, vb, va), jnp.where(gt, ib, ia)))
            if len(pairs) % 2:
                nxt.append(pairs[-1])
            pairs = nxt
        return pairs[0]

    def process_group(g, _):
        goff = g * _LANES
        rows = goff + lane
        best_vals = []
        best_idxs = []
        for _k in range(TOP_K):
            groups = []
            for c in range(0, NUM_EXPERTS, 8):
                leaves = []
                for e in range(c, c + 8):
                    v = tr[e, pl.ds(goff, _LANES)]
                    leaves.append((v, jnp.full((_LANES,), e, jnp.int32)))
                groups.append(argmax_tree(leaves))
            bv, bi = argmax_tree(groups)
            plsc.store_scatter(tr, [bi, rows], neg_inf)
            best_vals.append(bv)
            best_idxs.append(bi)
        exps = [jnp.exp(v - best_vals[0]) for v in best_vals]
        denom = exps[0]
        for e_ in exps[1:]:
            denom = denom + e_
        rows_k = rows * TOP_K
        for k in range(TOP_K):
            plsc.store_scatter(outv, [rows_k + k], exps[k] / denom)
            plsc.store_scatter(outi, [rows_k + k], best_idxs[k])
        return _

    def process_chunk(ci, _):
        base = wid * per_worker + ci * _SC_CHUNK
        pltpu.sync_copy(logits_t_hbm.at[:, pl.ds(base, _SC_CHUNK)], tr)
        lax.fori_loop(0, _SC_CHUNK // _LANES, process_group, None)
        pltpu.sync_copy(outv, vals_hbm.at[pl.ds(base * TOP_K,
                                                _SC_CHUNK * TOP_K)])
        pltpu.sync_copy(outi, idx_hbm.at[pl.ds(base * TOP_K,
                                               _SC_CHUNK * TOP_K)])
        return _

    lax.fori_loop(0, n_chunks, process_chunk, None)


def _sc_topk(logits_t):
    n_tokens = logits_t.shape[1]
    mesh = plsc.VectorSubcoreMesh(core_axis_name="c", subcore_axis_name="s")
    vals, idx = pl.kernel(
        _sc_topk_body,
        out_type=[
            jax.ShapeDtypeStruct((n_tokens * TOP_K,), jnp.float32),
            jax.ShapeDtypeStruct((n_tokens * TOP_K,), jnp.int32),
        ],
        mesh=mesh,
        compiler_params=pltpu.CompilerParams(needs_layout_passes=False),
        scratch_types=[
            pltpu.VMEM((NUM_EXPERTS, _SC_CHUNK), jnp.float32),
            pltpu.VMEM((_SC_CHUNK * TOP_K,), jnp.float32),
            pltpu.VMEM((_SC_CHUNK * TOP_K,), jnp.int32),
        ],
    )(logits_t)
    return vals.reshape(n_tokens, TOP_K), idx.reshape(n_tokens, TOP_K)


@jax.jit
def kernel(hidden_states, weight):
    h_flat = hidden_states.reshape(-1, hidden_states.shape[-1])
    wt = weight.T
    logits, logits_t = _tc_logits(h_flat, wt)
    vals, idx = _sc_topk(logits_t)
    return (logits, vals.astype(hidden_states.dtype), idx)
